# baseline (speedup 1.0000x reference)
import functools

import jax
import jax.numpy as jnp
from jax import lax
from jax.experimental import pallas as pl
from jax.experimental.pallas import tpu as pltpu
from jax.experimental.pallas import tpu_sc as plsc

D_MODEL = 1024
N_HEADS = 8
D_KEYS = 256
HALF = D_KEYS // 2
N_EXPERTS = 64
N_ROWS = N_EXPERTS * N_EXPERTS
TOP_K = 2
KK = N_HEADS * TOP_K
NEG = -1e30

SC_CORES = 2
SC_SUBCORES = 16
LANES = 16
NW = SC_CORES * SC_SUBCORES
CHUNKS = D_MODEL // LANES


def _top2(s, iota_e):
    v1 = jnp.max(s, axis=1, keepdims=True)
    i1 = jnp.min(jnp.where(s == v1, iota_e, N_EXPERTS), axis=1, keepdims=True)
    s_m = jnp.where(iota_e == i1, NEG, s)
    v2 = jnp.max(s_m, axis=1, keepdims=True)
    i2 = jnp.min(jnp.where(s_m == v2, iota_e, N_EXPERTS), axis=1, keepdims=True)
    return v1, i1, v2, i2


def _routing_body(q_ref, wq_ref, bq_ref, keys_ref, idx_ref, gate_ref):
    q = q_ref[...]
    qh = jax.lax.dot_general(q, wq_ref[...], (((1,), (1,)), ((), ())),
                             preferred_element_type=jnp.float32) + bq_ref[...]
    blk = q.shape[0]
    iota_e = jax.lax.broadcasted_iota(jnp.int32, (blk, N_EXPERTS), 1)
    idx_cols, gate_cols = [], []
    for h in range(N_HEADS):
        q1 = qh[:, h * D_KEYS:h * D_KEYS + HALF]
        q2 = qh[:, h * D_KEYS + HALF:(h + 1) * D_KEYS]
        s1 = jax.lax.dot_general(q1, keys_ref[2 * h], (((1,), (1,)), ((), ())),
                                 preferred_element_type=jnp.float32)
        s2 = jax.lax.dot_general(q2, keys_ref[2 * h + 1], (((1,), (1,)), ((), ())),
                                 preferred_element_type=jnp.float32)
        v1a, i1a, v1b, i1b = _top2(s1, iota_e)
        v2a, i2a, v2b, i2b = _top2(s2, iota_e)
        cv = [v1a + v2a, v1a + v2b, v1b + v2a, v1b + v2b]
        ci = [i1a * N_EXPERTS + i2a, i1a * N_EXPERTS + i2b,
              i1b * N_EXPERTS + i2a, i1b * N_EXPERTS + i2b]
        bv, bi, bp = cv[0], ci[0], jnp.zeros_like(ci[0])
        for j in range(1, 4):
            cond = cv[j] > bv
            bv = jnp.where(cond, cv[j], bv)
            bi = jnp.where(cond, ci[j], bi)
            bp = jnp.where(cond, j, bp)
        sv = jnp.full_like(bv, NEG)
        si = jnp.zeros_like(bi)
        for j in range(4):
            cond = (cv[j] > sv) & (bp != j)
            sv = jnp.where(cond, cv[j], sv)
            si = jnp.where(cond, ci[j], si)
        e = jnp.exp(sv - bv)
        g0 = 1.0 / (1.0 + e)
        g1 = e / (1.0 + e)
        idx_cols += [bi, si]
        gate_cols += [g0, g1]
    idx_ref[...] = jnp.concatenate(idx_cols, axis=1)
    gate_ref[...] = jnp.concatenate(gate_cols, axis=1)


def _routing(q_flat, W_q, bq_r, keys_r, interpret=False):
    NT = q_flat.shape[0]
    blk = 512
    return pl.pallas_call(
        _routing_body,
        grid=(NT // blk,),
        in_specs=[
            pl.BlockSpec((blk, D_MODEL), lambda i: (i, 0)),
            pl.BlockSpec((N_HEADS * D_KEYS, D_MODEL), lambda i: (0, 0)),
            pl.BlockSpec((1, N_HEADS * D_KEYS), lambda i: (0, 0)),
            pl.BlockSpec((2 * N_HEADS, N_EXPERTS, HALF), lambda i: (0, 0, 0)),
        ],
        out_specs=[
            pl.BlockSpec((blk, KK), lambda i: (i, 0)),
            pl.BlockSpec((blk, KK), lambda i: (i, 0)),
        ],
        out_shape=[
            jax.ShapeDtypeStruct((NT, KK), jnp.int32),
            jax.ShapeDtypeStruct((NT, KK), jnp.float32),
        ],
        interpret=interpret,
    )(q_flat, W_q, bq_r, keys_r)


def _gelu_tanh_vec(x):
    u = 0.7978845608028654 * (x + 0.044715 * (x * x * x))
    u = jnp.clip(u, -15.0, 15.0)
    t = 1.0 - 2.0 / (1.0 + jnp.exp(2.0 * u))
    return 0.5 * x * (1.0 + t)


TBLK = 4
NBUF = 4
GROUPS = D_MODEL // (2 * LANES)


def _prep_table(t):
    t3 = t.reshape(N_ROWS, GROUPS, 2, LANES).transpose(0, 1, 3, 2)
    tb = t3.astype(jnp.bfloat16).reshape(N_ROWS, D_MODEL // 2, 2)
    return jax.lax.bitcast_convert_type(tb, jnp.int32)


def _perm_q(q):
    NT = q.shape[0]
    return q.reshape(NT, GROUPS, LANES, 2).transpose(0, 1, 3, 2).reshape(NT, D_MODEL)


def _sc_combine_body(q_hbm, idx_hbm, gate_hbm, wd_hbm, wu_hbm, out_hbm,
                     idx_v, gate_v,
                     wdr0, wdr1, wdr2, wdr3, wur0, wur1, wur2, wur3,
                     qb0, qb1, ov0, ov1,
                     sd0, sd1, sd2, sd3, su0, su1, su2, su3,
                     sq0, sq1, so0, so1):
    sem_d = (sd0, sd1, sd2, sd3)
    sem_u = (su0, su1, su2, su3)
    tpw = idx_v.shape[0]
    nblk = tpw // TBLK
    wdrs = (wdr0, wdr1, wdr2, wdr3)
    wurs = (wur0, wur1, wur2, wur3)
    wid = lax.axis_index("s") * SC_CORES + lax.axis_index("c")
    base = wid * tpw
    pltpu.sync_copy(idx_hbm.at[pl.ds(base, tpw)], idx_v)
    pltpu.sync_copy(gate_hbm.at[pl.ds(base, tpw)], gate_v)

    def start_gather(t, j):
        idx_vec = idx_v[t]
        pltpu.async_copy(wd_hbm.at[idx_vec], wdrs[j], sem_d[j])
        pltpu.async_copy(wu_hbm.at[idx_vec], wurs[j], sem_u[j])

    def wait_gather(j):
        pltpu.make_async_copy(wd_hbm.at[pl.ds(0, KK)], wdrs[j], sem_d[j]).wait()
        pltpu.make_async_copy(wu_hbm.at[pl.ds(0, KK)], wurs[j], sem_u[j]).wait()

    def compute(t, lt, j, qc, ovc):
        wdr = wdrs[j]
        wur = wurs[j]
        pltpu.make_async_copy(wd_hbm.at[pl.ds(0, KK)], wdr, sem_d[j]).wait()

        def down_chunk(c, accs):
            qa = qc[lt, pl.ds(c * 2 * LANES, LANES)]
            qb = qc[lt, pl.ds(c * 2 * LANES + LANES, LANES)]
            out = []
            for r in range(KK):
                w32 = plsc.bitcast(wdr[r, pl.ds(c * LANES, LANES)],
                                   jnp.bfloat16)
                wa, wb = plsc.unpack(w32, format=plsc.PackFormat.INTERLEAVED)
                out.append(accs[r] + wa * qa + wb * qb)
            return tuple(out)

        zeros = tuple(jnp.zeros((LANES,), jnp.float32) for _ in range(KK))
        accs = lax.fori_loop(0, GROUPS, down_chunk, zeros, unroll=2)

        gv = gate_v[t]
        val_splats = []
        for r in range(KK):
            h_r = jnp.sum(accs[r])
            hs = jnp.full((LANES,), h_r)
            val_splats.append(_gelu_tanh_vec(hs) * gv[r])

        pltpu.make_async_copy(wu_hbm.at[pl.ds(0, KK)], wur, sem_u[j]).wait()

        def up_chunk(c, carry2):
            w32 = plsc.bitcast(wur[0, pl.ds(c * LANES, LANES)], jnp.bfloat16)
            wa, wb = plsc.unpack(w32, format=plsc.PackFormat.INTERLEAVED)
            acc_a = val_splats[0] * wa
            acc_b = val_splats[0] * wb
            for r in range(1, KK):
                w32 = plsc.bitcast(wur[r, pl.ds(c * LANES, LANES)],
                                   jnp.bfloat16)
                wa, wb = plsc.unpack(w32, format=plsc.PackFormat.INTERLEAVED)
                acc_a = acc_a + val_splats[r] * wa
                acc_b = acc_b + val_splats[r] * wb
            ovc[lt, pl.ds(c * 2 * LANES, LANES)] = acc_a
            ovc[lt, pl.ds(c * 2 * LANES + LANES, LANES)] = acc_b
            return carry2

        lax.fori_loop(0, GROUPS, up_chunk, 0, unroll=2)

    pltpu.async_copy(q_hbm.at[pl.ds(base, TBLK)], qb0, sq0)
    pltpu.async_copy(q_hbm.at[pl.ds(base + TBLK, TBLK)], qb1, sq1)
    for j in range(NBUF):
        start_gather(j, j)

    def super_body(sb, carry):
        for half in range(2):
            b = 2 * sb + half
            qc = (qb0, qb1)[half]
            ovc = (ov0, ov1)[half]
            sq = (sq0, sq1)[half]
            so = (so0, so1)[half]
            blk0 = b * TBLK
            pltpu.make_async_copy(q_hbm.at[pl.ds(0, TBLK)], qc, sq).wait()

            @pl.when(sb >= 1)
            def _():
                pltpu.make_async_copy(ovc, out_hbm.at[pl.ds(0, TBLK)], so).wait()

            def quad_body(g, carry2):
                for jj in range(NBUF):
                    lt = g * NBUF + jj
                    t = blk0 + lt
                    compute(t, lt, jj, qc, ovc)
                    nxt = jnp.minimum(t + NBUF, tpw - 1)
                    start_gather(nxt, jj)
                return carry2

            lax.fori_loop(0, TBLK // NBUF, quad_body, 0)
            pltpu.async_copy(ovc, out_hbm.at[pl.ds(base + blk0, TBLK)], so)
            nb = jnp.minimum(b + 2, nblk - 1)
            pltpu.async_copy(q_hbm.at[pl.ds(base + nb * TBLK, TBLK)], qc, sq)
        return carry

    lax.fori_loop(0, nblk // 2, super_body, 0)

    for j in range(NBUF):
        wait_gather(j)
    pltpu.make_async_copy(q_hbm.at[pl.ds(0, TBLK)], qb0, sq0).wait()
    pltpu.make_async_copy(q_hbm.at[pl.ds(0, TBLK)], qb1, sq1).wait()
    pltpu.make_async_copy(ov0, out_hbm.at[pl.ds(0, TBLK)], so0).wait()
    pltpu.make_async_copy(ov1, out_hbm.at[pl.ds(0, TBLK)], so1).wait()


def _sc_combine(q_perm, idx, gates, wd, wu):
    NT = q_perm.shape[0]
    tpw = NT // NW
    mesh = plsc.VectorSubcoreMesh(core_axis_name="c", subcore_axis_name="s")
    f = pl.kernel(
        _sc_combine_body,
        out_type=jax.ShapeDtypeStruct((NT, D_MODEL), jnp.float32),
        mesh=mesh,
        scratch_types=[
            pltpu.VMEM((tpw, KK), jnp.int32),
            pltpu.VMEM((tpw, KK), jnp.float32),
            pltpu.VMEM((KK, D_MODEL // 2), jnp.int32),
            pltpu.VMEM((KK, D_MODEL // 2), jnp.int32),
            pltpu.VMEM((KK, D_MODEL // 2), jnp.int32),
            pltpu.VMEM((KK, D_MODEL // 2), jnp.int32),
            pltpu.VMEM((KK, D_MODEL // 2), jnp.int32),
            pltpu.VMEM((KK, D_MODEL // 2), jnp.int32),
            pltpu.VMEM((KK, D_MODEL // 2), jnp.int32),
            pltpu.VMEM((KK, D_MODEL // 2), jnp.int32),
            pltpu.VMEM((TBLK, D_MODEL), jnp.float32),
            pltpu.VMEM((TBLK, D_MODEL), jnp.float32),
            pltpu.VMEM((TBLK, D_MODEL), jnp.float32),
            pltpu.VMEM((TBLK, D_MODEL), jnp.float32),
            pltpu.SemaphoreType.DMA,
            pltpu.SemaphoreType.DMA,
            pltpu.SemaphoreType.DMA,
            pltpu.SemaphoreType.DMA,
            pltpu.SemaphoreType.DMA,
            pltpu.SemaphoreType.DMA,
            pltpu.SemaphoreType.DMA,
            pltpu.SemaphoreType.DMA,
            pltpu.SemaphoreType.DMA,
            pltpu.SemaphoreType.DMA,
            pltpu.SemaphoreType.DMA,
            pltpu.SemaphoreType.DMA,
        ],
        compiler_params=pltpu.CompilerParams(needs_layout_passes=False),
    )
    return f(q_perm, idx, gates, wd, wu)


def _moe(queries, W_q, b_q, keys, w_down_embed, w_up_embed, interpret=False):
    B, T, D = queries.shape
    NT = B * T
    q_flat = queries.reshape(NT, D)
    keys_r = keys.reshape(2 * N_HEADS, N_EXPERTS, HALF)
    bq_r = b_q.reshape(1, N_HEADS * D_KEYS)
    idx, gates = _routing(q_flat, W_q, bq_r, keys_r, interpret=interpret)
    out = _sc_combine(q_flat, idx, gates,
                      _prep_table(w_down_embed), _prep_table(w_up_embed))
    return out.reshape(B, T, D)


def kernel(queries, W_q, b_q, keys, w_down_embed, w_up_embed):
    return _moe(queries, W_q, b_q, keys, w_down_embed, w_up_embed)


# speedup vs baseline: 1.1130x; 1.1130x over previous
import functools

import jax
import jax.numpy as jnp
from jax import lax
from jax.experimental import pallas as pl
from jax.experimental.pallas import tpu as pltpu
from jax.experimental.pallas import tpu_sc as plsc

D_MODEL = 1024
N_HEADS = 8
D_KEYS = 256
HALF = D_KEYS // 2
N_EXPERTS = 64
N_ROWS = N_EXPERTS * N_EXPERTS
TOP_K = 2
KK = N_HEADS * TOP_K
NEG = -1e30

SC_CORES = 2
SC_SUBCORES = 16
LANES = 16
NW = SC_CORES * SC_SUBCORES
CHUNKS = D_MODEL // LANES


def _top2(s, iota_e):
    v1 = jnp.max(s, axis=1, keepdims=True)
    i1 = jnp.min(jnp.where(s == v1, iota_e, N_EXPERTS), axis=1, keepdims=True)
    s_m = jnp.where(iota_e == i1, NEG, s)
    v2 = jnp.max(s_m, axis=1, keepdims=True)
    i2 = jnp.min(jnp.where(s_m == v2, iota_e, N_EXPERTS), axis=1, keepdims=True)
    return v1, i1, v2, i2


def _routing_body(q_ref, wq_ref, bq_ref, keys_ref, idx_ref, gate_ref):
    q = q_ref[...]
    qh = jax.lax.dot_general(q, wq_ref[...], (((1,), (1,)), ((), ())),
                             preferred_element_type=jnp.float32) + bq_ref[...]
    blk = q.shape[0]
    iota_e = jax.lax.broadcasted_iota(jnp.int32, (blk, N_EXPERTS), 1)
    idx_cols, gate_cols = [], []
    for h in range(N_HEADS):
        q1 = qh[:, h * D_KEYS:h * D_KEYS + HALF]
        q2 = qh[:, h * D_KEYS + HALF:(h + 1) * D_KEYS]
        s1 = jax.lax.dot_general(q1, keys_ref[2 * h], (((1,), (1,)), ((), ())),
                                 preferred_element_type=jnp.float32)
        s2 = jax.lax.dot_general(q2, keys_ref[2 * h + 1], (((1,), (1,)), ((), ())),
                                 preferred_element_type=jnp.float32)
        v1a, i1a, v1b, i1b = _top2(s1, iota_e)
        v2a, i2a, v2b, i2b = _top2(s2, iota_e)
        cv = [v1a + v2a, v1a + v2b, v1b + v2a, v1b + v2b]
        ci = [i1a * N_EXPERTS + i2a, i1a * N_EXPERTS + i2b,
              i1b * N_EXPERTS + i2a, i1b * N_EXPERTS + i2b]
        bv, bi, bp = cv[0], ci[0], jnp.zeros_like(ci[0])
        for j in range(1, 4):
            cond = cv[j] > bv
            bv = jnp.where(cond, cv[j], bv)
            bi = jnp.where(cond, ci[j], bi)
            bp = jnp.where(cond, j, bp)
        sv = jnp.full_like(bv, NEG)
        si = jnp.zeros_like(bi)
        for j in range(4):
            cond = (cv[j] > sv) & (bp != j)
            sv = jnp.where(cond, cv[j], sv)
            si = jnp.where(cond, ci[j], si)
        e = jnp.exp(sv - bv)
        g0 = 1.0 / (1.0 + e)
        g1 = e / (1.0 + e)
        idx_cols += [bi, si]
        gate_cols += [g0, g1]
    idx_ref[...] = jnp.concatenate(idx_cols, axis=1)
    gate_ref[...] = jnp.concatenate(gate_cols, axis=1)


def _routing(q_flat, W_q, bq_r, keys_r, interpret=False):
    NT = q_flat.shape[0]
    blk = 512
    return pl.pallas_call(
        _routing_body,
        grid=(NT // blk,),
        in_specs=[
            pl.BlockSpec((blk, D_MODEL), lambda i: (i, 0)),
            pl.BlockSpec((N_HEADS * D_KEYS, D_MODEL), lambda i: (0, 0)),
            pl.BlockSpec((1, N_HEADS * D_KEYS), lambda i: (0, 0)),
            pl.BlockSpec((2 * N_HEADS, N_EXPERTS, HALF), lambda i: (0, 0, 0)),
        ],
        out_specs=[
            pl.BlockSpec((blk, KK), lambda i: (i, 0)),
            pl.BlockSpec((blk, KK), lambda i: (i, 0)),
        ],
        out_shape=[
            jax.ShapeDtypeStruct((NT, KK), jnp.int32),
            jax.ShapeDtypeStruct((NT, KK), jnp.float32),
        ],
        interpret=interpret,
    )(q_flat, W_q, bq_r, keys_r)


def _gelu_tanh_vec(x):
    u = 0.7978845608028654 * (x + 0.044715 * (x * x * x))
    u = jnp.clip(u, -15.0, 15.0)
    t = 1.0 - 2.0 / (1.0 + jnp.exp(2.0 * u))
    return 0.5 * x * (1.0 + t)


TBLK = 8
NBUF = 4
GROUPS = D_MODEL // (2 * LANES)


def _prep_table(t):
    t3 = t.reshape(N_ROWS, GROUPS, 2, LANES).transpose(0, 1, 3, 2)
    tb = t3.astype(jnp.bfloat16).reshape(N_ROWS, D_MODEL // 2, 2)
    return jax.lax.bitcast_convert_type(tb, jnp.int32)


def _perm_q(q):
    NT = q.shape[0]
    return q.reshape(NT, GROUPS, LANES, 2).transpose(0, 1, 3, 2).reshape(NT, D_MODEL)


def _sc_combine_body(q_hbm, idx_hbm, gate_hbm, wd_hbm, wu_hbm, out_hbm,
                     idx_v, gate_v,
                     wdr0, wdr1, wdr2, wdr3, wur0, wur1, wur2, wur3,
                     qb0, qb1, ov0, ov1,
                     sd0, sd1, sd2, sd3, su0, su1, su2, su3,
                     sq0, sq1, so0, so1):
    sem_d = (sd0, sd1, sd2, sd3)
    sem_u = (su0, su1, su2, su3)
    tpw = idx_v.shape[0]
    nblk = tpw // TBLK
    wdrs = (wdr0, wdr1, wdr2, wdr3)
    wurs = (wur0, wur1, wur2, wur3)
    wid = lax.axis_index("s") * SC_CORES + lax.axis_index("c")
    base = wid * tpw
    pltpu.sync_copy(idx_hbm.at[pl.ds(base, tpw)], idx_v)
    pltpu.sync_copy(gate_hbm.at[pl.ds(base, tpw)], gate_v)

    def start_gather(t, j):
        idx_vec = idx_v[t]
        pltpu.async_copy(wd_hbm.at[idx_vec], wdrs[j], sem_d[j])
        pltpu.async_copy(wu_hbm.at[idx_vec], wurs[j], sem_u[j])

    def wait_gather(j):
        pltpu.make_async_copy(wd_hbm.at[pl.ds(0, KK)], wdrs[j], sem_d[j]).wait()
        pltpu.make_async_copy(wu_hbm.at[pl.ds(0, KK)], wurs[j], sem_u[j]).wait()

    def compute(t, lt, j, qc, ovc):
        wdr = wdrs[j]
        wur = wurs[j]
        pltpu.make_async_copy(wd_hbm.at[pl.ds(0, KK)], wdr, sem_d[j]).wait()

        def down_chunk(c, accs):
            qa = qc[lt, pl.ds(c * 2 * LANES, LANES)]
            qb = qc[lt, pl.ds(c * 2 * LANES + LANES, LANES)]
            out = []
            for r in range(KK):
                w32 = plsc.bitcast(wdr[r, pl.ds(c * LANES, LANES)],
                                   jnp.bfloat16)
                wa, wb = plsc.unpack(w32, format=plsc.PackFormat.INTERLEAVED)
                out.append(accs[r] + wa * qa + wb * qb)
            return tuple(out)

        zeros = tuple(jnp.zeros((LANES,), jnp.float32) for _ in range(KK))
        accs = lax.fori_loop(0, GROUPS, down_chunk, zeros)

        gv = gate_v[t]
        val_splats = []
        for r in range(KK):
            h_r = jnp.sum(accs[r])
            hs = jnp.full((LANES,), h_r)
            val_splats.append(_gelu_tanh_vec(hs) * gv[r])

        pltpu.make_async_copy(wu_hbm.at[pl.ds(0, KK)], wur, sem_u[j]).wait()

        def up_chunk(c, carry2):
            w32 = plsc.bitcast(wur[0, pl.ds(c * LANES, LANES)], jnp.bfloat16)
            wa, wb = plsc.unpack(w32, format=plsc.PackFormat.INTERLEAVED)
            acc_a = val_splats[0] * wa
            acc_b = val_splats[0] * wb
            for r in range(1, KK):
                w32 = plsc.bitcast(wur[r, pl.ds(c * LANES, LANES)],
                                   jnp.bfloat16)
                wa, wb = plsc.unpack(w32, format=plsc.PackFormat.INTERLEAVED)
                acc_a = acc_a + val_splats[r] * wa
                acc_b = acc_b + val_splats[r] * wb
            ovc[lt, pl.ds(c * 2 * LANES, LANES)] = acc_a
            ovc[lt, pl.ds(c * 2 * LANES + LANES, LANES)] = acc_b
            return carry2

        lax.fori_loop(0, GROUPS, up_chunk, 0)

    pltpu.async_copy(q_hbm.at[pl.ds(base, TBLK)], qb0, sq0)
    pltpu.async_copy(q_hbm.at[pl.ds(base + TBLK, TBLK)], qb1, sq1)
    for j in range(NBUF):
        start_gather(j, j)

    def super_body(sb, carry):
        for half in range(2):
            b = 2 * sb + half
            qc = (qb0, qb1)[half]
            ovc = (ov0, ov1)[half]
            sq = (sq0, sq1)[half]
            so = (so0, so1)[half]
            blk0 = b * TBLK
            pltpu.make_async_copy(q_hbm.at[pl.ds(0, TBLK)], qc, sq).wait()

            @pl.when(sb >= 1)
            def _():
                pltpu.make_async_copy(ovc, out_hbm.at[pl.ds(0, TBLK)], so).wait()

            def quad_body(g, carry2):
                for jj in range(NBUF):
                    lt = g * NBUF + jj
                    t = blk0 + lt
                    compute(t, lt, jj, qc, ovc)
                    nxt = jnp.minimum(t + NBUF, tpw - 1)
                    start_gather(nxt, jj)
                return carry2

            lax.fori_loop(0, TBLK // NBUF, quad_body, 0)
            pltpu.async_copy(ovc, out_hbm.at[pl.ds(base + blk0, TBLK)], so)
            nb = jnp.minimum(b + 2, nblk - 1)
            pltpu.async_copy(q_hbm.at[pl.ds(base + nb * TBLK, TBLK)], qc, sq)
        return carry

    lax.fori_loop(0, nblk // 2, super_body, 0)

    for j in range(NBUF):
        wait_gather(j)
    pltpu.make_async_copy(q_hbm.at[pl.ds(0, TBLK)], qb0, sq0).wait()
    pltpu.make_async_copy(q_hbm.at[pl.ds(0, TBLK)], qb1, sq1).wait()
    pltpu.make_async_copy(ov0, out_hbm.at[pl.ds(0, TBLK)], so0).wait()
    pltpu.make_async_copy(ov1, out_hbm.at[pl.ds(0, TBLK)], so1).wait()


def _sc_combine(q_perm, idx, gates, wd, wu):
    NT = q_perm.shape[0]
    tpw = NT // NW
    mesh = plsc.VectorSubcoreMesh(core_axis_name="c", subcore_axis_name="s")
    f = pl.kernel(
        _sc_combine_body,
        out_type=jax.ShapeDtypeStruct((NT, D_MODEL), jnp.float32),
        mesh=mesh,
        scratch_types=[
            pltpu.VMEM((tpw, KK), jnp.int32),
            pltpu.VMEM((tpw, KK), jnp.float32),
            pltpu.VMEM((KK, D_MODEL // 2), jnp.int32),
            pltpu.VMEM((KK, D_MODEL // 2), jnp.int32),
            pltpu.VMEM((KK, D_MODEL // 2), jnp.int32),
            pltpu.VMEM((KK, D_MODEL // 2), jnp.int32),
            pltpu.VMEM((KK, D_MODEL // 2), jnp.int32),
            pltpu.VMEM((KK, D_MODEL // 2), jnp.int32),
            pltpu.VMEM((KK, D_MODEL // 2), jnp.int32),
            pltpu.VMEM((KK, D_MODEL // 2), jnp.int32),
            pltpu.VMEM((TBLK, D_MODEL), jnp.float32),
            pltpu.VMEM((TBLK, D_MODEL), jnp.float32),
            pltpu.VMEM((TBLK, D_MODEL), jnp.float32),
            pltpu.VMEM((TBLK, D_MODEL), jnp.float32),
            pltpu.SemaphoreType.DMA,
            pltpu.SemaphoreType.DMA,
            pltpu.SemaphoreType.DMA,
            pltpu.SemaphoreType.DMA,
            pltpu.SemaphoreType.DMA,
            pltpu.SemaphoreType.DMA,
            pltpu.SemaphoreType.DMA,
            pltpu.SemaphoreType.DMA,
            pltpu.SemaphoreType.DMA,
            pltpu.SemaphoreType.DMA,
            pltpu.SemaphoreType.DMA,
            pltpu.SemaphoreType.DMA,
        ],
        compiler_params=pltpu.CompilerParams(needs_layout_passes=False),
    )
    return f(q_perm, idx, gates, wd, wu)


def _moe(queries, W_q, b_q, keys, w_down_embed, w_up_embed, interpret=False):
    B, T, D = queries.shape
    NT = B * T
    q_flat = queries.reshape(NT, D)
    keys_r = keys.reshape(2 * N_HEADS, N_EXPERTS, HALF)
    bq_r = b_q.reshape(1, N_HEADS * D_KEYS)
    wd_p = _prep_table(w_down_embed)
    wu_p = _prep_table(w_up_embed)
    halves = []
    for qh_flat in (q_flat[:NT // 2], q_flat[NT // 2:]):
        idx, gates = _routing(qh_flat, W_q, bq_r, keys_r, interpret=interpret)
        halves.append(_sc_combine(qh_flat, idx, gates, wd_p, wu_p))
    out = jnp.concatenate(halves, axis=0)
    return out.reshape(B, T, D)


def kernel(queries, W_q, b_q, keys, w_down_embed, w_up_embed):
    return _moe(queries, W_q, b_q, keys, w_down_embed, w_up_embed)
